# Initial kernel scaffold; baseline (speedup 1.0000x reference)
import jax
import jax.numpy as jnp
from jax.experimental import pallas as pl


def kernel(z, w_attn, b_attn, flat_idx, segment_ids):
    raise NotImplementedError("write your pallas kernel here")


# speedup vs baseline: 3.6722x; 3.6722x over previous
import functools

import numpy as np

import jax
import jax.numpy as jnp
from jax import lax
from jax.experimental import pallas as pl
from jax.experimental.pallas import tpu as pltpu
from jax.experimental.pallas import tpu_sc as plsc

_NEG = np.float32(-3.0e38)


def _mv_body(z_ref, w_ref, y_ref):
    y_ref[...] = jnp.sum(z_ref[...] * w_ref[...], axis=1)[None, None, :]


def _matvec(z, w):
    n, dim = z.shape
    blk = 1024
    grid = n // blk
    y3d = pl.pallas_call(
        _mv_body,
        grid=(grid,),
        in_specs=[
            pl.BlockSpec((blk, dim), lambda k: (k, 0)),
            pl.BlockSpec((1, dim), lambda k: (0, 0)),
        ],
        out_specs=pl.BlockSpec((1, 1, blk), lambda k: (k, 0, 0)),
        out_shape=jax.ShapeDtypeStruct((grid, 1, blk), jnp.float32),
    )(z, w.reshape(1, dim))
    return y3d.reshape(n)


def _make_sc_kernel(m, n, num_segments):
    mesh = plsc.VectorSubcoreMesh(core_axis_name="c", subcore_axis_name="s")

    @functools.partial(
        pl.kernel,
        out_type=jax.ShapeDtypeStruct((2, num_segments, n), jnp.float32),
        mesh=mesh,
        compiler_params=pltpu.CompilerParams(needs_layout_passes=False),
        scratch_types=[
            pltpu.VMEM((m,), jnp.int32),
            pltpu.VMEM((m + 16,), jnp.int32),
            pltpu.VMEM((m,), jnp.float32),
            pltpu.VMEM((n,), jnp.float32),
        ],
    )
    def sc_kernel(y_hbm, idx_hbm, seg_hbm, s_out, seg_v, idx_v, y_v, srow_v):
        c = lax.axis_index("c")
        t = lax.axis_index("s")
        pltpu.sync_copy(seg_hbm, seg_v)
        pltpu.sync_copy(idx_hbm, idx_v.at[pl.ds(0, m)])
        pltpu.sync_copy(y_hbm, y_v)
        lanes = lax.iota(jnp.int32, 16)

        def cnt_body(k, carry):
            s_acc, e_acc = carry
            v = seg_v[pl.ds(k * 16, 16)]
            one = jnp.float32(1.0)
            nil = jnp.float32(0.0)
            s_acc = s_acc + jnp.where(v < t, one, nil)
            e_acc = e_acc + jnp.where(v <= t, one, nil)
            return s_acc, e_acc

        zero16 = jnp.zeros((16,), jnp.float32)
        s_acc, e_acc = lax.fori_loop(0, m // 16, cnt_body, (zero16, zero16))
        start = jnp.sum(s_acc).astype(jnp.int32)
        end = jnp.sum(e_acc).astype(jnp.int32)

        def zero_body(k, _):
            srow_v[pl.ds(k * 16, 16)] = jnp.zeros((16,), jnp.float32)
            return 0

        lax.fori_loop(0, n // 16, zero_body, 0)

        def mx_body(i, mv):
            pos = start + i * 16
            valid = (lanes + pos) < end
            iv = idx_v[pl.ds(pos, 16)]
            lv = plsc.load_gather(y_v, [iv], mask=valid)
            return jnp.maximum(mv, jnp.where(valid, lv, _NEG))

        nchunks = (end - start + 15) // 16
        mvec = lax.fori_loop(0, nchunks, mx_body, jnp.full((16,), _NEG, jnp.float32))
        m_t = jnp.max(mvec)

        def dn_body(i, dv):
            pos = start + i * 16
            valid = (lanes + pos) < end
            iv = idx_v[pl.ds(pos, 16)]
            lv = plsc.load_gather(y_v, [iv], mask=valid)
            e = jnp.where(valid, jnp.exp(lv - m_t), jnp.float32(0.0))
            return dv + e

        dvec = lax.fori_loop(0, nchunks, dn_body, jnp.zeros((16,), jnp.float32))
        denom = jnp.sum(dvec)
        denom_vec = jnp.full((16,), 1.0, jnp.float32) * jnp.where(
            denom == 0.0, jnp.float32(1.0), denom
        )
        inv_vec = jnp.full((16,), 1.0, jnp.float32) / denom_vec

        mid = (start + end) // 2
        h0 = jnp.where(c == 0, start, mid)
        h1 = jnp.where(c == 0, mid, end)

        def sc_body(i, _):
            pos = h0 + i * 16
            valid = (lanes + pos) < h1
            iv = idx_v[pl.ds(pos, 16)]
            lv = plsc.load_gather(y_v, [iv], mask=valid)
            w = jnp.where(valid, jnp.exp(lv - m_t) * inv_vec, jnp.float32(0.0))
            plsc.addupdate_scatter(srow_v, [iv], w, mask=valid)
            return 0

        hchunks = (h1 - h0 + 15) // 16
        lax.fori_loop(0, hchunks, sc_body, 0)

        pltpu.sync_copy(srow_v, s_out.at[c, t])

    return sc_kernel


def _mm_body(s2_ref, z_ref, out_ref):
    k = pl.program_id(0)
    s_blk = s2_ref[0] + s2_ref[1]
    part = jnp.dot(s_blk, z_ref[...], preferred_element_type=jnp.float32)

    @pl.when(k == 0)
    def _():
        out_ref[...] = part

    @pl.when(k > 0)
    def _():
        out_ref[...] += part


def _weighted_matmul(s2, z, num_segments):
    n, dim = z.shape
    blk = 1024
    grid = n // blk
    return pl.pallas_call(
        _mm_body,
        grid=(grid,),
        in_specs=[
            pl.BlockSpec((2, num_segments, blk), lambda k: (0, 0, k)),
            pl.BlockSpec((blk, dim), lambda k: (k, 0)),
        ],
        out_specs=pl.BlockSpec((num_segments, dim), lambda k: (0, 0)),
        out_shape=jax.ShapeDtypeStruct((num_segments, dim), jnp.float32),
    )(s2, z)


def kernel(z, w_attn, b_attn, flat_idx, segment_ids):
    del b_attn
    n, dim = z.shape
    (m,) = flat_idx.shape
    num_segments = 16
    idx32 = flat_idx.astype(jnp.int32)
    seg32 = segment_ids.astype(jnp.int32)
    y = _matvec(z, w_attn)
    s2 = _make_sc_kernel(m, n, num_segments)(y, idx32, seg32)
    return _weighted_matmul(s2, z, num_segments)


# speedup vs baseline: 4.2709x; 1.1630x over previous
import functools

import numpy as np

import jax
import jax.numpy as jnp
from jax import lax
from jax.experimental import pallas as pl
from jax.experimental.pallas import tpu as pltpu
from jax.experimental.pallas import tpu_sc as plsc

_NEG = np.float32(-3.0e38)


def _mv_body(z_ref, w_ref, y_ref):
    y_ref[...] = jnp.sum(z_ref[...] * w_ref[...], axis=1)[None, None, :]


def _matvec(z, w):
    n, dim = z.shape
    blk = 2048
    grid = n // blk
    y3d = pl.pallas_call(
        _mv_body,
        grid=(grid,),
        in_specs=[
            pl.BlockSpec((blk, dim), lambda k: (k, 0)),
            pl.BlockSpec((1, dim), lambda k: (0, 0)),
        ],
        out_specs=pl.BlockSpec((1, 1, blk), lambda k: (k, 0, 0)),
        out_shape=jax.ShapeDtypeStruct((grid, 1, blk), jnp.float32),
    )(z, w.reshape(1, dim))
    return y3d.reshape(n)


def _make_sc_kernel(m, n, num_segments):
    mesh = plsc.VectorSubcoreMesh(core_axis_name="c", subcore_axis_name="s")

    @functools.partial(
        pl.kernel,
        out_type=[
            jax.ShapeDtypeStruct((2, num_segments, n), jnp.float32),
            jax.ShapeDtypeStruct((2, num_segments, 32), jnp.float32),
        ],
        mesh=mesh,
        compiler_params=pltpu.CompilerParams(needs_layout_passes=False),
        scratch_types=[
            pltpu.VMEM((m,), jnp.int32),
            pltpu.VMEM((m + 32,), jnp.int32),
            pltpu.VMEM((m,), jnp.float32),
            pltpu.VMEM((n,), jnp.float32),
            pltpu.VMEM((32,), jnp.float32),
        ],
    )
    def sc_kernel(y_hbm, idx_hbm, seg_hbm, s_out, aux_out, seg_v, idx_v, y_v, srow_v, aux_v):
        c = lax.axis_index("c")
        t = lax.axis_index("s")
        pltpu.sync_copy(seg_hbm, seg_v)
        pltpu.sync_copy(idx_hbm, idx_v.at[pl.ds(0, m)])
        pltpu.sync_copy(y_hbm, y_v)
        lanes = lax.iota(jnp.int32, 16)
        one = jnp.float32(1.0)
        nil = jnp.float32(0.0)
        zf16 = jnp.zeros((16,), jnp.float32)

        def cz_body(k, carry):
            s_acc, e_acc = carry
            v = seg_v[pl.ds(k * 16, 16)]
            srow_v[pl.ds(k * 16, 16)] = zf16
            s_acc = s_acc + jnp.where(v < t, one, nil)
            e_acc = e_acc + jnp.where(v <= t, one, nil)
            return s_acc, e_acc

        assert m == n and m % 16 == 0
        s_acc, e_acc = lax.fori_loop(0, m // 16, cz_body, (zf16, zf16), unroll=8)
        start = jnp.sum(s_acc).astype(jnp.int32)
        end = jnp.sum(e_acc).astype(jnp.int32)

        mid = (start + end) // 2
        h0 = jnp.where(c == 0, start, mid)
        h1 = jnp.where(c == 0, mid, end)
        nch = (h1 - h0 + 31) // 32

        def mx_body(i, mv):
            pos = h0 + i * 32
            mv0, mv1 = mv
            valid0 = (lanes + pos) < h1
            valid1 = (lanes + (pos + 16)) < h1
            iv0 = idx_v[pl.ds(pos, 16)]
            iv1 = idx_v[pl.ds(pos + 16, 16)]
            lv0 = plsc.load_gather(y_v, [iv0], mask=valid0)
            lv1 = plsc.load_gather(y_v, [iv1], mask=valid1)
            mv0 = jnp.maximum(mv0, jnp.where(valid0, lv0, _NEG))
            mv1 = jnp.maximum(mv1, jnp.where(valid1, lv1, _NEG))
            return mv0, mv1

        neg16 = jnp.full((16,), _NEG, jnp.float32)
        mv0, mv1 = lax.fori_loop(0, nch, mx_body, (neg16, neg16))
        m_c = jnp.max(jnp.maximum(mv0, mv1))

        def sc_body(i, dv):
            pos = h0 + i * 32
            dv0, dv1 = dv
            valid0 = (lanes + pos) < h1
            valid1 = (lanes + (pos + 16)) < h1
            iv0 = idx_v[pl.ds(pos, 16)]
            iv1 = idx_v[pl.ds(pos + 16, 16)]
            lv0 = plsc.load_gather(y_v, [iv0], mask=valid0)
            lv1 = plsc.load_gather(y_v, [iv1], mask=valid1)
            e0 = jnp.where(valid0, jnp.exp(lv0 - m_c), nil)
            e1 = jnp.where(valid1, jnp.exp(lv1 - m_c), nil)
            plsc.addupdate_scatter(srow_v, [iv0], e0, mask=valid0)
            plsc.addupdate_scatter(srow_v, [iv1], e1, mask=valid1)
            return dv0 + e0, dv1 + e1

        dv0, dv1 = lax.fori_loop(0, nch, sc_body, (zf16, zf16))
        d_c = jnp.sum(dv0 + dv1)

        ones16 = jnp.full((16,), 1.0, jnp.float32)
        aux_v[pl.ds(0, 16)] = ones16 * m_c
        aux_v[pl.ds(16, 16)] = ones16 * d_c
        pltpu.sync_copy(srow_v, s_out.at[c, t])
        pltpu.sync_copy(aux_v, aux_out.at[c, t])

    return sc_kernel


def _mm_body(s2_ref, aux_ref, z_ref, out_ref):
    k = pl.program_id(0)
    m0 = aux_ref[0, :, 0:1]
    m1 = aux_ref[1, :, 0:1]
    mm = jnp.maximum(m0, m1)
    a0 = jnp.exp(m0 - mm)
    a1 = jnp.exp(m1 - mm)
    s_blk = a0 * s2_ref[0] + a1 * s2_ref[1]
    part = jnp.dot(s_blk, z_ref[...], preferred_element_type=jnp.float32)

    @pl.when(k == 0)
    def _():
        out_ref[...] = part

    @pl.when(k > 0)
    def _():
        out_ref[...] += part

    @pl.when(k == pl.num_programs(0) - 1)
    def _():
        d = a0 * aux_ref[0, :, 16:17] + a1 * aux_ref[1, :, 16:17]
        d = jnp.where(d == 0.0, jnp.float32(1.0), d)
        out_ref[...] = out_ref[...] / d

    return


def _weighted_matmul(s2, aux, z, num_segments):
    n, dim = z.shape
    blk = 2048
    grid = n // blk
    return pl.pallas_call(
        _mm_body,
        grid=(grid,),
        in_specs=[
            pl.BlockSpec((2, num_segments, blk), lambda k: (0, 0, k)),
            pl.BlockSpec((2, num_segments, 32), lambda k: (0, 0, 0)),
            pl.BlockSpec((blk, dim), lambda k: (k, 0)),
        ],
        out_specs=pl.BlockSpec((num_segments, dim), lambda k: (0, 0)),
        out_shape=jax.ShapeDtypeStruct((num_segments, dim), jnp.float32),
    )(s2, aux, z)


def kernel(z, w_attn, b_attn, flat_idx, segment_ids):
    del b_attn
    n, dim = z.shape
    (m,) = flat_idx.shape
    num_segments = 16
    idx32 = flat_idx.astype(jnp.int32)
    seg32 = segment_ids.astype(jnp.int32)
    y = _matvec(z, w_attn)
    s2, aux = _make_sc_kernel(m, n, num_segments)(y, idx32, seg32)
    return _weighted_matmul(s2, aux, z, num_segments)
